# baseline (speedup 1.0000x reference)
import functools

import jax
import jax.numpy as jnp
from jax import lax
from jax.experimental import pallas as pl
from jax.experimental.pallas import tpu as pltpu
from jax.experimental.pallas import tpu_sc as plsc

B = 4096
E = 1000
D = 16
LANES = 16
NC = 2
NS = 16
NW = NC * NS
RPW = B // NW
CHUNK = 16
NFULL = E // LANES
TAIL = E - NFULL * LANES


def _sc_body(counts_hbm, emb_hbm, out_hbm, emb_v, cnt_v, out_v):
    wid = lax.axis_index("s") * NC + lax.axis_index("c")
    base = wid * RPW
    pltpu.sync_copy(emb_hbm, emb_v)

    def do_chunk(c, carry):
        row0 = base + c * CHUNK
        pltpu.sync_copy(counts_hbm.at[pl.ds(row0 * E, CHUNK * E)], cnt_v)

        def ec_step(ec, accs):
            e0 = ec * LANES
            evecs = [emb_v[pl.ds((e0 + l) * D, D)] for l in range(LANES)]
            new = []
            for j in range(CHUNK):
                cf = cnt_v[pl.ds(j * E + e0, LANES)].astype(jnp.float32)
                a = accs[j]
                for l in range(LANES):
                    a = a + cf[l] * evecs[l]
                new.append(a)
            return tuple(new)

        accs = tuple(jnp.zeros((D,), jnp.float32) for _ in range(CHUNK))
        accs = lax.fori_loop(0, NFULL, ec_step, accs)

        e0 = E - LANES
        evecs = [emb_v[pl.ds((e0 + l) * D, D)] for l in range(LANES - TAIL, LANES)]
        for j in range(CHUNK):
            cf = cnt_v[pl.ds(j * E + e0, LANES)].astype(jnp.float32)
            a = accs[j]
            for i, l in enumerate(range(LANES - TAIL, LANES)):
                a = a + cf[l] * evecs[i]
            out_v[pl.ds((c * CHUNK + j) * D, D)] = a
        return carry

    lax.fori_loop(0, RPW // CHUNK, do_chunk, 0)
    pltpu.sync_copy(out_v, out_hbm.at[pl.ds(base * D, RPW * D)])


@functools.partial(jax.jit, static_argnames=())
def kernel(element_counts, emb):
    mesh = plsc.VectorSubcoreMesh(core_axis_name="c", subcore_axis_name="s")
    sc_fn = pl.kernel(
        _sc_body,
        out_type=jax.ShapeDtypeStruct((B * D,), jnp.float32),
        mesh=mesh,
        scratch_types=[
            pltpu.VMEM((E * D,), jnp.float32),
            pltpu.VMEM((CHUNK * E,), jnp.int32),
            pltpu.VMEM((RPW * D,), jnp.float32),
        ],
    )
    out = sc_fn(element_counts.reshape(B * E), emb.reshape(E * D))
    return out.reshape(B, D)


# speedup vs baseline: 1.3736x; 1.3736x over previous
import functools

import jax
import jax.numpy as jnp
from jax import lax
from jax.experimental import pallas as pl
from jax.experimental.pallas import tpu as pltpu
from jax.experimental.pallas import tpu_sc as plsc

B = 4096
E = 1000
D = 16
LANES = 16
NC = 2
NS = 16
NW = NC * NS
RPW = B // NW
CHUNK = 32
JGRP = 8
NFULL = E // LANES
TAIL = E - NFULL * LANES


def _sc_body(counts_hbm, emb_hbm, out_hbm, emb_v, cnt_v, out_v):
    wid = lax.axis_index("s") * NC + lax.axis_index("c")
    base = wid * RPW
    pltpu.sync_copy(emb_hbm, emb_v)

    def do_chunk(c, carry):
        row0 = base + c * CHUNK
        pltpu.sync_copy(counts_hbm.at[pl.ds(row0 * E, CHUNK * E)], cnt_v)

        for g in range(CHUNK // JGRP):
            def ec_step(ec, accs, g=g):
                e0 = ec * LANES
                evecs = [emb_v[pl.ds((e0 + l) * D, D)] for l in range(LANES)]
                new = []
                for j in range(JGRP):
                    row = g * JGRP + j
                    cf = cnt_v[pl.ds(row * E + e0, LANES)].astype(jnp.float32)
                    a = accs[j]
                    for l in range(LANES):
                        a = a + cf[l] * evecs[l]
                    new.append(a)
                return tuple(new)

            accs = tuple(jnp.zeros((D,), jnp.float32) for _ in range(JGRP))
            accs = lax.fori_loop(0, NFULL, ec_step, accs)

            e0 = E - LANES
            evecs = [emb_v[pl.ds((e0 + l) * D, D)]
                     for l in range(LANES - TAIL, LANES)]
            for j in range(JGRP):
                row = g * JGRP + j
                cf = cnt_v[pl.ds(row * E + e0, LANES)].astype(jnp.float32)
                a = accs[j]
                for i, l in enumerate(range(LANES - TAIL, LANES)):
                    a = a + cf[l] * evecs[i]
                out_v[pl.ds((c * CHUNK + row) * D, D)] = a
        return carry

    lax.fori_loop(0, RPW // CHUNK, do_chunk, 0)
    pltpu.sync_copy(out_v, out_hbm.at[pl.ds(base * D, RPW * D)])


@functools.partial(jax.jit, static_argnames=())
def kernel(element_counts, emb):
    mesh = plsc.VectorSubcoreMesh(core_axis_name="c", subcore_axis_name="s")
    sc_fn = pl.kernel(
        _sc_body,
        out_type=jax.ShapeDtypeStruct((B * D,), jnp.float32),
        mesh=mesh,
        scratch_types=[
            pltpu.VMEM((E * D,), jnp.float32),
            pltpu.VMEM((CHUNK * E,), jnp.int32),
            pltpu.VMEM((RPW * D,), jnp.float32),
        ],
    )
    out = sc_fn(element_counts.reshape(B * E), emb.reshape(E * D))
    return out.reshape(B, D)
